# baseline (speedup 1.0000x reference)
import functools
import jax
import jax.numpy as jnp
from jax import lax
from jax.experimental import pallas as pl
from jax.experimental.pallas import tpu as pltpu
from jax.experimental.pallas import tpu_sc as plsc

_TT = 256


def _scores_body(q_ref, k_ref, wq_ref, wk_ref, ls_ref, out_ref, m_ref):
    h = pl.program_id(1)
    t = pl.program_id(2)
    T, D = k_ref.shape[2], k_ref.shape[3]
    NB = T // D
    S = wq_ref.shape[2]
    BS = D

    @pl.when(t == 0)
    def _():
        k = k_ref[0, 0]
        ks = jnp.mean(k.reshape(NB, BS, D), axis=1)
        kr = jnp.dot(ks, wk_ref[0], preferred_element_type=jnp.float32)
        ls = ls_ref[...]
        hmask = (jax.lax.broadcasted_iota(jnp.int32, ls.shape, 0) == h)
        ls_h = jnp.sum(jnp.where(hmask, ls, 0.0))
        m_ref[...] = kr * jnp.exp(ls_h)

    q = q_ref[0, 0]
    qr = jnp.dot(q, wq_ref[0], preferred_element_type=jnp.float32)
    s = jax.lax.dot_general(m_ref[...], qr, (((1,), (1,)), ((), ())),
                            preferred_element_type=jnp.float32)

    vv = jax.lax.broadcasted_iota(jnp.int32, (NB, _TT), 0)
    tb = (t * _TT + jax.lax.broadcasted_iota(jnp.int32, (NB, _TT), 1)) // BS
    s = jnp.where(vv > tb, -1e30 * (vv + 1).astype(jnp.float32), s)
    out_ref[0, 0] = s


def _tc_scores(Q, K, Wq, Wk, logit_scale):
    B, T, HQ, D = Q.shape
    H = K.shape[2]
    G = HQ // H
    S = Wq.shape[2]
    NB = T // D

    Qrep = jnp.transpose(Q[:, :, ::G, :], (0, 2, 1, 3))
    Kt = jnp.transpose(K, (0, 2, 1, 3))
    ls2 = logit_scale.reshape(H, 1).astype(jnp.float32)

    s = pl.pallas_call(
        _scores_body,
        grid=(B, H, T // _TT),
        in_specs=[
            pl.BlockSpec((1, 1, _TT, D), lambda b, h, t: (b, h, t, 0)),
            pl.BlockSpec((1, 1, T, D), lambda b, h, t: (b, h, 0, 0)),
            pl.BlockSpec((1, D, S), lambda b, h, t: (h, 0, 0)),
            pl.BlockSpec((1, D, S), lambda b, h, t: (h, 0, 0)),
            pl.BlockSpec((H, 1), lambda b, h, t: (0, 0)),
        ],
        out_specs=pl.BlockSpec((1, 1, NB, _TT), lambda b, h, t: (b, h, 0, t)),
        out_shape=jax.ShapeDtypeStruct((B, H, NB, T), jnp.float32),
        scratch_shapes=[pltpu.VMEM((NB, S), jnp.float32)],
        compiler_params=pltpu.CompilerParams(
            dimension_semantics=("parallel", "parallel", "arbitrary")),
    )(Qrep, Kt, Wq, Wk, ls2)
    return jnp.transpose(s, (0, 1, 3, 2))


def _merge16(ka, va, kb, vb):
    kbr = lax.rev(kb, (0,))
    vbr = lax.rev(vb, (0,))
    m = ka >= kbr
    kc = jnp.where(m, ka, kbr)
    vc = jnp.where(m, va, vbr)
    return plsc.sort_key_val(kc, vc, descending=True)


def _sc_router(scores2d, T, NB, S):
    R = scores2d.shape[0]
    info = plsc.get_sparse_core_info()
    NC, NS, L = info.num_cores, info.num_subcores, info.num_lanes
    NW = NC * NS
    rows_per_w = R // NW
    CH = 512
    n_chunks = rows_per_w // CH
    mesh = plsc.VectorSubcoreMesh(core_axis_name="c", subcore_axis_name="s")

    @functools.partial(
        pl.kernel, mesh=mesh,
        out_type=jax.ShapeDtypeStruct((R, S), jnp.int32),
        scratch_types=[
            pltpu.VMEM((CH, NB), jnp.float32),
            pltpu.VMEM((CH, S), jnp.int32),
        ],
        compiler_params=pltpu.CompilerParams(needs_layout_passes=False),
    )
    def k(s_hbm, out_hbm, sv, ov):
        wid = lax.axis_index("s") * NC + lax.axis_index("c")
        iota = lax.iota(jnp.int32, L)

        def chunk_body(ci, _):
            base = wid * rows_per_w + ci * CH
            pltpu.sync_copy(s_hbm.at[pl.ds(base, CH)], sv)

            def row_body(i, __):
                grow = base + i
                t = grow % T
                tb = t // (T // NB)
                lb0 = tb
                lb1 = jnp.maximum(tb - 1, 0)
                ks = []
                vs = []
                for c4 in range(NB // L):
                    kk, vv = plsc.sort_key_val(
                        sv[i, pl.ds(c4 * L, L)], iota + c4 * L,
                        descending=True)
                    ks.append(kk)
                    vs.append(vv)
                k01, v01 = _merge16(ks[0], vs[0], ks[1], vs[1])
                k23, v23 = _merge16(ks[2], vs[2], ks[3], vs[3])
                _, win = _merge16(k01, v01, k23, v23)
                si, _unused = plsc.sort_key_val(win, win)
                q1 = plsc.all_reduce_population_count(si < lb1)
                q2 = plsc.all_reduce_population_count(si < lb0) + 1
                sh = jnp.where(iota > q1, 1, 0) + jnp.where(iota > q2, 1, 0)
                jw = iota - sh
                g = lax.gather(
                    si, jw[:, None],
                    lax.GatherDimensionNumbers(offset_dims=(),
                                               collapsed_slice_dims=(0,),
                                               start_index_map=(0,)),
                    (1,), mode=lax.GatherScatterMode.PROMISE_IN_BOUNDS)
                res = jnp.where(iota == q1, lb1,
                                jnp.where(iota == q2, lb0, g))
                ov[i, :] = res
                return __

            lax.fori_loop(0, CH, row_body, 0)
            pltpu.sync_copy(ov, out_hbm.at[pl.ds(base, CH)])
            return _

        lax.fori_loop(0, n_chunks, chunk_body, 0)

    return k


def kernel(Q, K, Wq, Wk, logit_scale, block_size, selected_blocks, groups):
    B, T, HQ, D = Q.shape
    H = K.shape[2]
    S = Wq.shape[2]
    NB = T // D

    s = _tc_scores(Q, K, Wq, Wk, logit_scale)
    s2 = s.reshape(B * H * T, NB)
    out2 = _sc_router(s2, T, NB, S)(s2)
    out = out2.reshape(B, H, T, S)
    return jnp.transpose(out, (0, 2, 1, 3))


# speedup vs baseline: 1.0027x; 1.0027x over previous
import functools
import jax
import jax.numpy as jnp
from jax import lax
from jax.experimental import pallas as pl
from jax.experimental.pallas import tpu as pltpu
from jax.experimental.pallas import tpu_sc as plsc

_TT = 256


def _scores_body(q_ref, k_ref, wq_ref, wk_ref, ls_ref, out_ref, m_ref):
    h = pl.program_id(1)
    t = pl.program_id(2)
    T, D = k_ref.shape[2], k_ref.shape[3]
    NB = T // D
    S = wq_ref.shape[2]
    BS = D

    @pl.when(t == 0)
    def _():
        k = k_ref[0, 0]
        ks = jnp.mean(k.reshape(NB, BS, D), axis=1)
        kr = jnp.dot(ks, wk_ref[0], preferred_element_type=jnp.float32)
        ls = ls_ref[...]
        hmask = (jax.lax.broadcasted_iota(jnp.int32, ls.shape, 0) == h)
        ls_h = jnp.sum(jnp.where(hmask, ls, 0.0))
        m_ref[...] = kr * jnp.exp(ls_h)

    q = q_ref[0, 0]
    qr = jnp.dot(q, wq_ref[0], preferred_element_type=jnp.float32)
    s = jax.lax.dot_general(m_ref[...], qr, (((1,), (1,)), ((), ())),
                            preferred_element_type=jnp.float32)

    vv = jax.lax.broadcasted_iota(jnp.int32, (NB, _TT), 0)
    tb = (t * _TT + jax.lax.broadcasted_iota(jnp.int32, (NB, _TT), 1)) // BS
    s = jnp.where(vv > tb, -1e30 * (vv + 1).astype(jnp.float32), s)
    out_ref[0, 0] = s.T


def _tc_scores(Q, K, Wq, Wk, logit_scale):
    B, T, HQ, D = Q.shape
    H = K.shape[2]
    G = HQ // H
    S = Wq.shape[2]
    NB = T // D

    Qrep = jnp.transpose(Q[:, :, ::G, :], (0, 2, 1, 3))
    Kt = jnp.transpose(K, (0, 2, 1, 3))
    ls2 = logit_scale.reshape(H, 1).astype(jnp.float32)

    s = pl.pallas_call(
        _scores_body,
        grid=(B, H, T // _TT),
        in_specs=[
            pl.BlockSpec((1, 1, _TT, D), lambda b, h, t: (b, h, t, 0)),
            pl.BlockSpec((1, 1, T, D), lambda b, h, t: (b, h, 0, 0)),
            pl.BlockSpec((1, D, S), lambda b, h, t: (h, 0, 0)),
            pl.BlockSpec((1, D, S), lambda b, h, t: (h, 0, 0)),
            pl.BlockSpec((H, 1), lambda b, h, t: (0, 0)),
        ],
        out_specs=pl.BlockSpec((1, 1, _TT, NB), lambda b, h, t: (b, h, t, 0)),
        out_shape=jax.ShapeDtypeStruct((B, H, T, NB), jnp.float32),
        scratch_shapes=[pltpu.VMEM((NB, S), jnp.float32)],
        compiler_params=pltpu.CompilerParams(
            dimension_semantics=("parallel", "parallel", "arbitrary")),
    )(Qrep, Kt, Wq, Wk, ls2)
    return s


def _merge16(ka, va, kb, vb):
    kbr = lax.rev(kb, (0,))
    vbr = lax.rev(vb, (0,))
    m = ka >= kbr
    kc = jnp.where(m, ka, kbr)
    vc = jnp.where(m, va, vbr)
    return plsc.sort_key_val(kc, vc, descending=True)


def _sc_router(scores2d, T, NB, S):
    R = scores2d.shape[0]
    info = plsc.get_sparse_core_info()
    NC, NS, L = info.num_cores, info.num_subcores, info.num_lanes
    NW = NC * NS
    rows_per_w = R // NW
    CH = 512
    n_chunks = rows_per_w // CH
    mesh = plsc.VectorSubcoreMesh(core_axis_name="c", subcore_axis_name="s")

    @functools.partial(
        pl.kernel, mesh=mesh,
        out_type=jax.ShapeDtypeStruct((R, S), jnp.int32),
        scratch_types=[
            pltpu.VMEM((CH, NB), jnp.float32),
            pltpu.VMEM((CH, S), jnp.int32),
        ],
        compiler_params=pltpu.CompilerParams(needs_layout_passes=False),
    )
    def k(s_hbm, out_hbm, sv, ov):
        wid = lax.axis_index("s") * NC + lax.axis_index("c")
        iota = lax.iota(jnp.int32, L)

        def chunk_body(ci, _):
            base = wid * rows_per_w + ci * CH
            pltpu.sync_copy(s_hbm.at[pl.ds(base, CH)], sv)

            def row_body(i, __):
                grow = base + i
                t = grow % T
                tb = t // (T // NB)
                lb0 = tb
                lb1 = jnp.maximum(tb - 1, 0)
                ks = []
                vs = []
                for c4 in range(NB // L):
                    kk, vv = plsc.sort_key_val(
                        sv[i, pl.ds(c4 * L, L)], iota + c4 * L,
                        descending=True)
                    ks.append(kk)
                    vs.append(vv)
                k01, v01 = _merge16(ks[0], vs[0], ks[1], vs[1])
                k23, v23 = _merge16(ks[2], vs[2], ks[3], vs[3])
                _, win = _merge16(k01, v01, k23, v23)
                si, _unused = plsc.sort_key_val(win, win)
                q1 = plsc.all_reduce_population_count(si < lb1)
                q2 = plsc.all_reduce_population_count(si < lb0) + 1
                sh = jnp.where(iota > q1, 1, 0) + jnp.where(iota > q2, 1, 0)
                jw = iota - sh
                g = lax.gather(
                    si, jw[:, None],
                    lax.GatherDimensionNumbers(offset_dims=(),
                                               collapsed_slice_dims=(0,),
                                               start_index_map=(0,)),
                    (1,), mode=lax.GatherScatterMode.PROMISE_IN_BOUNDS)
                res = jnp.where(iota == q1, lb1,
                                jnp.where(iota == q2, lb0, g))
                ov[i, :] = res
                return __

            lax.fori_loop(0, CH, row_body, 0, unroll=4)
            pltpu.sync_copy(ov, out_hbm.at[pl.ds(base, CH)])
            return _

        lax.fori_loop(0, n_chunks, chunk_body, 0)

    return k


def kernel(Q, K, Wq, Wk, logit_scale, block_size, selected_blocks, groups):
    B, T, HQ, D = Q.shape
    H = K.shape[2]
    S = Wq.shape[2]
    NB = T // D

    s = _tc_scores(Q, K, Wq, Wk, logit_scale)
    s2 = s.reshape(B * H * T, NB)
    out2 = _sc_router(s2, T, NB, S)(s2)
    out = out2.reshape(B, H, T, S)
    return jnp.transpose(out, (0, 2, 1, 3))


# speedup vs baseline: 1.2429x; 1.2396x over previous
import functools
import jax
import jax.numpy as jnp
from jax import lax
from jax.experimental import pallas as pl
from jax.experimental.pallas import tpu as pltpu
from jax.experimental.pallas import tpu_sc as plsc

_TT = 512


def _scores_body(q_ref, k_ref, wq_ref, wk_ref, ls_ref, out_ref, m_ref):
    h = pl.program_id(1)
    t = pl.program_id(2)
    T, D = k_ref.shape[2], k_ref.shape[3]
    NB = T // D
    S = wq_ref.shape[2]
    BS = D

    @pl.when(t == 0)
    def _():
        k = k_ref[0, 0]
        ks = jnp.mean(k.reshape(NB, BS, D), axis=1)
        kr = jnp.dot(ks, wk_ref[0], preferred_element_type=jnp.float32)
        ls = ls_ref[...]
        hmask = (jax.lax.broadcasted_iota(jnp.int32, ls.shape, 0) == h)
        ls_h = jnp.sum(jnp.where(hmask, ls, 0.0))
        m_ref[...] = kr * jnp.exp(ls_h)

    q = q_ref[0, 0]
    qr = jnp.dot(q, wq_ref[0], preferred_element_type=jnp.float32)
    s = jax.lax.dot_general(m_ref[...], qr, (((1,), (1,)), ((), ())),
                            preferred_element_type=jnp.float32)

    vv = jax.lax.broadcasted_iota(jnp.int32, (NB, _TT), 0)
    tb = (t * _TT + jax.lax.broadcasted_iota(jnp.int32, (NB, _TT), 1)) // BS
    s = jnp.where(vv > tb, -1e30 * (vv + 1).astype(jnp.float32), s)
    out_ref[0, 0] = s.T


def _tc_scores(Q, K, Wq, Wk, logit_scale):
    B, T, HQ, D = Q.shape
    H = K.shape[2]
    G = HQ // H
    S = Wq.shape[2]
    NB = T // D

    Qrep = jnp.transpose(Q[:, :, ::G, :], (0, 2, 1, 3))
    Kt = jnp.transpose(K, (0, 2, 1, 3))
    ls2 = logit_scale.reshape(H, 1).astype(jnp.float32)

    s = pl.pallas_call(
        _scores_body,
        grid=(B, H, T // _TT),
        in_specs=[
            pl.BlockSpec((1, 1, _TT, D), lambda b, h, t: (b, h, t, 0)),
            pl.BlockSpec((1, 1, T, D), lambda b, h, t: (b, h, 0, 0)),
            pl.BlockSpec((1, D, S), lambda b, h, t: (h, 0, 0)),
            pl.BlockSpec((1, D, S), lambda b, h, t: (h, 0, 0)),
            pl.BlockSpec((H, 1), lambda b, h, t: (0, 0)),
        ],
        out_specs=pl.BlockSpec((1, 1, _TT, NB), lambda b, h, t: (b, h, t, 0)),
        out_shape=jax.ShapeDtypeStruct((B, H, T, NB), jnp.float32),
        scratch_shapes=[pltpu.VMEM((NB, S), jnp.float32)],
        compiler_params=pltpu.CompilerParams(
            dimension_semantics=("parallel", "parallel", "arbitrary")),
    )(Qrep, Kt, Wq, Wk, ls2)
    return s


def _merge16(ka, va, kb, vb):
    kbr = lax.rev(kb, (0,))
    vbr = lax.rev(vb, (0,))
    m = ka >= kbr
    kc = jnp.where(m, ka, kbr)
    vc = jnp.where(m, va, vbr)
    return plsc.sort_key_val(kc, vc, descending=True)


def _sc_router(scores2d, T, NB, S):
    R = scores2d.shape[0]
    info = plsc.get_sparse_core_info()
    NC, NS, L = info.num_cores, info.num_subcores, info.num_lanes
    NW = NC * NS
    rows_per_w = R // NW
    CH = 512
    n_chunks = rows_per_w // CH
    mesh = plsc.VectorSubcoreMesh(core_axis_name="c", subcore_axis_name="s")

    @functools.partial(
        pl.kernel, mesh=mesh,
        out_type=jax.ShapeDtypeStruct((R, S), jnp.int32),
        scratch_types=[
            pltpu.VMEM((CH, NB), jnp.float32),
            pltpu.VMEM((CH, S), jnp.int32),
        ],
        compiler_params=pltpu.CompilerParams(needs_layout_passes=False),
    )
    def k(s_hbm, out_hbm, sv, ov):
        wid = lax.axis_index("s") * NC + lax.axis_index("c")
        iota = lax.iota(jnp.int32, L)

        def chunk_body(ci, _):
            base = wid * rows_per_w + ci * CH
            pltpu.sync_copy(s_hbm.at[pl.ds(base, CH)], sv)

            def row_body(i, __):
                grow = base + i
                t = grow % T
                tb = t // (T // NB)
                lb0 = tb
                lb1 = jnp.maximum(tb - 1, 0)
                ks = []
                vs = []
                for c4 in range(NB // L):
                    kk, vv = plsc.sort_key_val(
                        sv[i, pl.ds(c4 * L, L)], iota + c4 * L,
                        descending=True)
                    ks.append(kk)
                    vs.append(vv)
                k01, v01 = _merge16(ks[0], vs[0], ks[1], vs[1])
                k23, v23 = _merge16(ks[2], vs[2], ks[3], vs[3])
                _, win = _merge16(k01, v01, k23, v23)
                si, _unused = plsc.sort_key_val(win, win)
                q1 = plsc.all_reduce_population_count(si < lb1)
                q2 = plsc.all_reduce_population_count(si < lb0) + 1
                sh = jnp.where(iota > q1, 1, 0) + jnp.where(iota > q2, 1, 0)
                jw = iota - sh
                g = lax.gather(
                    si, jw[:, None],
                    lax.GatherDimensionNumbers(offset_dims=(),
                                               collapsed_slice_dims=(0,),
                                               start_index_map=(0,)),
                    (1,), mode=lax.GatherScatterMode.PROMISE_IN_BOUNDS)
                res = jnp.where(iota == q1, lb1,
                                jnp.where(iota == q2, lb0, g))
                ov[i, :] = res
                return __

            lax.fori_loop(0, CH, row_body, 0, unroll=4)
            pltpu.sync_copy(ov, out_hbm.at[pl.ds(base, CH)])
            return _

        lax.fori_loop(0, n_chunks, chunk_body, 0)

    return k


def kernel(Q, K, Wq, Wk, logit_scale, block_size, selected_blocks, groups):
    B, T, HQ, D = Q.shape
    H = K.shape[2]
    S = Wq.shape[2]
    NB = T // D

    s = _tc_scores(Q, K, Wq, Wk, logit_scale)
    s2 = s.reshape(B * H * T, NB)
    out2 = _sc_router(s2, T, NB, S)(s2)
    out = out2.reshape(B, H, T, S)
    return jnp.transpose(out, (0, 2, 1, 3))


# speedup vs baseline: 1.4105x; 1.1349x over previous
import functools
import jax
import jax.numpy as jnp
from jax import lax
from jax.experimental import pallas as pl
from jax.experimental.pallas import tpu as pltpu
from jax.experimental.pallas import tpu_sc as plsc

_TT = 1024


def _scores_body(q_ref, k_ref, wq_ref, wk_ref, ls_ref, out_ref, m_ref):
    h = pl.program_id(1)
    t = pl.program_id(2)
    T, D = k_ref.shape[2], k_ref.shape[3]
    NB = T // D
    S = wq_ref.shape[2]
    BS = D

    @pl.when(t == 0)
    def _():
        k = k_ref[0, 0]
        ks = jnp.mean(k.reshape(NB, BS, D), axis=1)
        kr = jnp.dot(ks, wk_ref[0], preferred_element_type=jnp.float32)
        ls = ls_ref[...]
        hmask = (jax.lax.broadcasted_iota(jnp.int32, ls.shape, 0) == h)
        ls_h = jnp.sum(jnp.where(hmask, ls, 0.0))
        m_ref[...] = kr * jnp.exp(ls_h)

    q = q_ref[0, 0]
    qr = jnp.dot(q, wq_ref[0], preferred_element_type=jnp.float32)
    s = jax.lax.dot_general(m_ref[...], qr, (((1,), (1,)), ((), ())),
                            preferred_element_type=jnp.float32)

    vv = jax.lax.broadcasted_iota(jnp.int32, (NB, _TT), 0)
    tb = (t * _TT + jax.lax.broadcasted_iota(jnp.int32, (NB, _TT), 1)) // BS
    s = jnp.where(vv > tb, -1e30 * (vv + 1).astype(jnp.float32), s)
    out_ref[0, 0] = s.T


def _tc_scores(Q, K, Wq, Wk, logit_scale):
    B, T, HQ, D = Q.shape
    H = K.shape[2]
    G = HQ // H
    S = Wq.shape[2]
    NB = T // D

    Qrep = jnp.transpose(Q[:, :, ::G, :], (0, 2, 1, 3))
    Kt = jnp.transpose(K, (0, 2, 1, 3))
    ls2 = logit_scale.reshape(H, 1).astype(jnp.float32)

    s = pl.pallas_call(
        _scores_body,
        grid=(B, H, T // _TT),
        in_specs=[
            pl.BlockSpec((1, 1, _TT, D), lambda b, h, t: (b, h, t, 0)),
            pl.BlockSpec((1, 1, T, D), lambda b, h, t: (b, h, 0, 0)),
            pl.BlockSpec((1, D, S), lambda b, h, t: (h, 0, 0)),
            pl.BlockSpec((1, D, S), lambda b, h, t: (h, 0, 0)),
            pl.BlockSpec((H, 1), lambda b, h, t: (0, 0)),
        ],
        out_specs=pl.BlockSpec((1, 1, _TT, NB), lambda b, h, t: (b, h, t, 0)),
        out_shape=jax.ShapeDtypeStruct((B, H, T, NB), jnp.float32),
        scratch_shapes=[pltpu.VMEM((NB, S), jnp.float32)],
        compiler_params=pltpu.CompilerParams(
            dimension_semantics=("parallel", "parallel", "arbitrary")),
    )(Qrep, Kt, Wq, Wk, ls2)
    return s


def _merge16(ka, va, kb, vb):
    kbr = lax.rev(kb, (0,))
    vbr = lax.rev(vb, (0,))
    m = ka >= kbr
    kc = jnp.where(m, ka, kbr)
    vc = jnp.where(m, va, vbr)
    return plsc.sort_key_val(kc, vc, descending=True)


def _sc_router(scores2d, T, NB, S):
    R = scores2d.shape[0]
    info = plsc.get_sparse_core_info()
    NC, NS, L = info.num_cores, info.num_subcores, info.num_lanes
    NW = NC * NS
    rows_per_w = R // NW
    CH = 512
    n_chunks = rows_per_w // CH
    mesh = plsc.VectorSubcoreMesh(core_axis_name="c", subcore_axis_name="s")

    @functools.partial(
        pl.kernel, mesh=mesh,
        out_type=jax.ShapeDtypeStruct((R, S), jnp.int32),
        scratch_types=[
            pltpu.VMEM((CH, NB), jnp.float32),
            pltpu.VMEM((CH, S), jnp.int32),
        ],
        compiler_params=pltpu.CompilerParams(needs_layout_passes=False),
    )
    def k(s_hbm, out_hbm, sv, ov):
        wid = lax.axis_index("s") * NC + lax.axis_index("c")
        iota = lax.iota(jnp.int32, L)

        def chunk_body(ci, _):
            base = wid * rows_per_w + ci * CH
            pltpu.sync_copy(s_hbm.at[pl.ds(base, CH)], sv)

            def row_body(i, __):
                grow = base + i
                t = grow % T
                tb = t // (T // NB)
                lb0 = tb
                lb1 = jnp.maximum(tb - 1, 0)
                ks = []
                vs = []
                for c4 in range(NB // L):
                    kk, vv = plsc.sort_key_val(
                        sv[i, pl.ds(c4 * L, L)], iota + c4 * L,
                        descending=True)
                    ks.append(kk)
                    vs.append(vv)
                k01, v01 = _merge16(ks[0], vs[0], ks[1], vs[1])
                k23, v23 = _merge16(ks[2], vs[2], ks[3], vs[3])
                _, win = _merge16(k01, v01, k23, v23)
                si, _unused = plsc.sort_key_val(win, win)
                q1 = plsc.all_reduce_population_count(si < lb1)
                q2 = plsc.all_reduce_population_count(si < lb0) + 1
                sh = jnp.where(iota > q1, 1, 0) + jnp.where(iota > q2, 1, 0)
                jw = iota - sh
                g = lax.gather(
                    si, jw[:, None],
                    lax.GatherDimensionNumbers(offset_dims=(),
                                               collapsed_slice_dims=(0,),
                                               start_index_map=(0,)),
                    (1,), mode=lax.GatherScatterMode.PROMISE_IN_BOUNDS)
                res = jnp.where(iota == q1, lb1,
                                jnp.where(iota == q2, lb0, g))
                ov[i, :] = res
                return __

            lax.fori_loop(0, CH, row_body, 0, unroll=4)
            pltpu.sync_copy(ov, out_hbm.at[pl.ds(base, CH)])
            return _

        lax.fori_loop(0, n_chunks, chunk_body, 0)

    return k


def kernel(Q, K, Wq, Wk, logit_scale, block_size, selected_blocks, groups):
    B, T, HQ, D = Q.shape
    H = K.shape[2]
    S = Wq.shape[2]
    NB = T // D

    s = _tc_scores(Q, K, Wq, Wk, logit_scale)
    s2 = s.reshape(B * H * T, NB)
    out2 = _sc_router(s2, T, NB, S)(s2)
    out = out2.reshape(B, H, T, S)
    return jnp.transpose(out, (0, 2, 1, 3))


# speedup vs baseline: 1.5187x; 1.0767x over previous
import functools
import jax
import jax.numpy as jnp
from jax import lax
from jax.experimental import pallas as pl
from jax.experimental.pallas import tpu as pltpu
from jax.experimental.pallas import tpu_sc as plsc

_TT = 2048


def _scores_body(q_ref, k_ref, wq_ref, wk_ref, ls_ref, out_ref, m_ref):
    h = pl.program_id(1)
    t = pl.program_id(2)
    T, D = k_ref.shape[2], k_ref.shape[3]
    NB = T // D
    S = wq_ref.shape[2]
    BS = D

    @pl.when(t == 0)
    def _():
        k = k_ref[0, 0]
        ks = jnp.mean(k.reshape(NB, BS, D), axis=1)
        kr = jnp.dot(ks, wk_ref[0], preferred_element_type=jnp.float32)
        ls = ls_ref[...]
        hmask = (jax.lax.broadcasted_iota(jnp.int32, ls.shape, 0) == h)
        ls_h = jnp.sum(jnp.where(hmask, ls, 0.0))
        m_ref[...] = kr * jnp.exp(ls_h)

    q = q_ref[0, 0]
    qr = jnp.dot(q, wq_ref[0], preferred_element_type=jnp.float32)
    s = jax.lax.dot_general(m_ref[...], qr, (((1,), (1,)), ((), ())),
                            preferred_element_type=jnp.float32)

    vv = jax.lax.broadcasted_iota(jnp.int32, (NB, _TT), 0)
    tb = (t * _TT + jax.lax.broadcasted_iota(jnp.int32, (NB, _TT), 1)) // BS
    s = jnp.where(vv > tb, -1e30 * (vv + 1).astype(jnp.float32), s)
    out_ref[0, 0] = s.T


def _tc_scores(Q, K, Wq, Wk, logit_scale):
    B, T, HQ, D = Q.shape
    H = K.shape[2]
    G = HQ // H
    S = Wq.shape[2]
    NB = T // D

    Qrep = jnp.transpose(Q[:, :, ::G, :], (0, 2, 1, 3))
    Kt = jnp.transpose(K, (0, 2, 1, 3))
    ls2 = logit_scale.reshape(H, 1).astype(jnp.float32)

    s = pl.pallas_call(
        _scores_body,
        grid=(B, H, T // _TT),
        in_specs=[
            pl.BlockSpec((1, 1, _TT, D), lambda b, h, t: (b, h, t, 0)),
            pl.BlockSpec((1, 1, T, D), lambda b, h, t: (b, h, 0, 0)),
            pl.BlockSpec((1, D, S), lambda b, h, t: (h, 0, 0)),
            pl.BlockSpec((1, D, S), lambda b, h, t: (h, 0, 0)),
            pl.BlockSpec((H, 1), lambda b, h, t: (0, 0)),
        ],
        out_specs=pl.BlockSpec((1, 1, _TT, NB), lambda b, h, t: (b, h, t, 0)),
        out_shape=jax.ShapeDtypeStruct((B, H, T, NB), jnp.float32),
        scratch_shapes=[pltpu.VMEM((NB, S), jnp.float32)],
        compiler_params=pltpu.CompilerParams(
            dimension_semantics=("parallel", "parallel", "arbitrary")),
    )(Qrep, Kt, Wq, Wk, ls2)
    return s


def _merge16(ka, va, kb, vb):
    kbr = lax.rev(kb, (0,))
    vbr = lax.rev(vb, (0,))
    m = ka >= kbr
    kc = jnp.where(m, ka, kbr)
    vc = jnp.where(m, va, vbr)
    return plsc.sort_key_val(kc, vc, descending=True)


def _sc_router(scores2d, T, NB, S):
    R = scores2d.shape[0]
    info = plsc.get_sparse_core_info()
    NC, NS, L = info.num_cores, info.num_subcores, info.num_lanes
    NW = NC * NS
    rows_per_w = R // NW
    CH = 512
    n_chunks = rows_per_w // CH
    mesh = plsc.VectorSubcoreMesh(core_axis_name="c", subcore_axis_name="s")

    @functools.partial(
        pl.kernel, mesh=mesh,
        out_type=jax.ShapeDtypeStruct((R, S), jnp.int32),
        scratch_types=[
            pltpu.VMEM((CH, NB), jnp.float32),
            pltpu.VMEM((CH, S), jnp.int32),
        ],
        compiler_params=pltpu.CompilerParams(needs_layout_passes=False),
    )
    def k(s_hbm, out_hbm, sv, ov):
        wid = lax.axis_index("s") * NC + lax.axis_index("c")
        iota = lax.iota(jnp.int32, L)

        def chunk_body(ci, _):
            base = wid * rows_per_w + ci * CH
            pltpu.sync_copy(s_hbm.at[pl.ds(base, CH)], sv)

            def row_body(i, __):
                grow = base + i
                t = grow % T
                tb = t // (T // NB)
                lb0 = tb
                lb1 = jnp.maximum(tb - 1, 0)
                ks = []
                vs = []
                for c4 in range(NB // L):
                    kk, vv = plsc.sort_key_val(
                        sv[i, pl.ds(c4 * L, L)], iota + c4 * L,
                        descending=True)
                    ks.append(kk)
                    vs.append(vv)
                k01, v01 = _merge16(ks[0], vs[0], ks[1], vs[1])
                k23, v23 = _merge16(ks[2], vs[2], ks[3], vs[3])
                _, win = _merge16(k01, v01, k23, v23)
                si, _unused = plsc.sort_key_val(win, win)
                q1 = plsc.all_reduce_population_count(si < lb1)
                q2 = plsc.all_reduce_population_count(si < lb0) + 1
                sh = jnp.where(iota > q1, 1, 0) + jnp.where(iota > q2, 1, 0)
                jw = iota - sh
                g = lax.gather(
                    si, jw[:, None],
                    lax.GatherDimensionNumbers(offset_dims=(),
                                               collapsed_slice_dims=(0,),
                                               start_index_map=(0,)),
                    (1,), mode=lax.GatherScatterMode.PROMISE_IN_BOUNDS)
                res = jnp.where(iota == q1, lb1,
                                jnp.where(iota == q2, lb0, g))
                ov[i, :] = res
                return __

            lax.fori_loop(0, CH, row_body, 0, unroll=4)
            pltpu.sync_copy(ov, out_hbm.at[pl.ds(base, CH)])
            return _

        lax.fori_loop(0, n_chunks, chunk_body, 0)

    return k


def kernel(Q, K, Wq, Wk, logit_scale, block_size, selected_blocks, groups):
    B, T, HQ, D = Q.shape
    H = K.shape[2]
    S = Wq.shape[2]
    NB = T // D

    s = _tc_scores(Q, K, Wq, Wk, logit_scale)
    s2 = s.reshape(B * H * T, NB)
    out2 = _sc_router(s2, T, NB, S)(s2)
    out = out2.reshape(B, H, T, S)
    return jnp.transpose(out, (0, 2, 1, 3))


# speedup vs baseline: 1.6450x; 1.0832x over previous
import functools
import jax
import jax.numpy as jnp
from jax import lax
from jax.experimental import pallas as pl
from jax.experimental.pallas import tpu as pltpu
from jax.experimental.pallas import tpu_sc as plsc

_TT = 4096


def _scores_body(q_ref, k_ref, wq_ref, wk_ref, ls_ref, out_ref, m_ref):
    h = pl.program_id(1)
    t = pl.program_id(2)
    T, D = k_ref.shape[2], k_ref.shape[3]
    NB = T // D
    S = wq_ref.shape[2]
    BS = D

    @pl.when(t == 0)
    def _():
        k = k_ref[0, 0]
        ks = jnp.mean(k.reshape(NB, BS, D), axis=1)
        kr = jnp.dot(ks, wk_ref[0], preferred_element_type=jnp.float32)
        ls = ls_ref[...]
        hmask = (jax.lax.broadcasted_iota(jnp.int32, ls.shape, 0) == h)
        ls_h = jnp.sum(jnp.where(hmask, ls, 0.0))
        m_ref[...] = kr * jnp.exp(ls_h)

    q = q_ref[0, 0]
    qr = jnp.dot(q, wq_ref[0], preferred_element_type=jnp.float32)
    s = jax.lax.dot_general(m_ref[...], qr, (((1,), (1,)), ((), ())),
                            preferred_element_type=jnp.float32)

    vv = jax.lax.broadcasted_iota(jnp.int32, (NB, _TT), 0)
    tb = (t * _TT + jax.lax.broadcasted_iota(jnp.int32, (NB, _TT), 1)) // BS
    s = jnp.where(vv > tb, -1e30 * (vv + 1).astype(jnp.float32), s)
    out_ref[0, 0] = s.T


def _tc_scores(Q, K, Wq, Wk, logit_scale):
    B, T, HQ, D = Q.shape
    H = K.shape[2]
    G = HQ // H
    S = Wq.shape[2]
    NB = T // D

    Qrep = jnp.transpose(Q[:, :, ::G, :], (0, 2, 1, 3))
    Kt = jnp.transpose(K, (0, 2, 1, 3))
    ls2 = logit_scale.reshape(H, 1).astype(jnp.float32)

    s = pl.pallas_call(
        _scores_body,
        grid=(B, H, T // _TT),
        in_specs=[
            pl.BlockSpec((1, 1, _TT, D), lambda b, h, t: (b, h, t, 0)),
            pl.BlockSpec((1, 1, T, D), lambda b, h, t: (b, h, 0, 0)),
            pl.BlockSpec((1, D, S), lambda b, h, t: (h, 0, 0)),
            pl.BlockSpec((1, D, S), lambda b, h, t: (h, 0, 0)),
            pl.BlockSpec((H, 1), lambda b, h, t: (0, 0)),
        ],
        out_specs=pl.BlockSpec((1, 1, _TT, NB), lambda b, h, t: (b, h, t, 0)),
        out_shape=jax.ShapeDtypeStruct((B, H, T, NB), jnp.float32),
        scratch_shapes=[pltpu.VMEM((NB, S), jnp.float32)],
        compiler_params=pltpu.CompilerParams(
            dimension_semantics=("parallel", "parallel", "arbitrary")),
    )(Qrep, Kt, Wq, Wk, ls2)
    return s


def _merge16(ka, va, kb, vb):
    kbr = lax.rev(kb, (0,))
    vbr = lax.rev(vb, (0,))
    m = ka >= kbr
    kc = jnp.where(m, ka, kbr)
    vc = jnp.where(m, va, vbr)
    return plsc.sort_key_val(kc, vc, descending=True)


def _sc_router(scores2d, T, NB, S):
    R = scores2d.shape[0]
    info = plsc.get_sparse_core_info()
    NC, NS, L = info.num_cores, info.num_subcores, info.num_lanes
    NW = NC * NS
    rows_per_w = R // NW
    CH = 512
    n_chunks = rows_per_w // CH
    mesh = plsc.VectorSubcoreMesh(core_axis_name="c", subcore_axis_name="s")

    @functools.partial(
        pl.kernel, mesh=mesh,
        out_type=jax.ShapeDtypeStruct((R, S), jnp.int32),
        scratch_types=[
            pltpu.VMEM((CH, NB), jnp.float32),
            pltpu.VMEM((CH, S), jnp.int32),
        ],
        compiler_params=pltpu.CompilerParams(needs_layout_passes=False),
    )
    def k(s_hbm, out_hbm, sv, ov):
        wid = lax.axis_index("s") * NC + lax.axis_index("c")
        iota = lax.iota(jnp.int32, L)

        def chunk_body(ci, _):
            base = wid * rows_per_w + ci * CH
            pltpu.sync_copy(s_hbm.at[pl.ds(base, CH)], sv)

            def row_body(i, __):
                grow = base + i
                t = grow % T
                tb = t // (T // NB)
                lb0 = tb
                lb1 = jnp.maximum(tb - 1, 0)
                ks = []
                vs = []
                for c4 in range(NB // L):
                    kk, vv = plsc.sort_key_val(
                        sv[i, pl.ds(c4 * L, L)], iota + c4 * L,
                        descending=True)
                    ks.append(kk)
                    vs.append(vv)
                k01, v01 = _merge16(ks[0], vs[0], ks[1], vs[1])
                k23, v23 = _merge16(ks[2], vs[2], ks[3], vs[3])
                _, win = _merge16(k01, v01, k23, v23)
                si, _unused = plsc.sort_key_val(win, win)
                q1 = plsc.all_reduce_population_count(si < lb1)
                q2 = plsc.all_reduce_population_count(si < lb0) + 1
                sh = jnp.where(iota > q1, 1, 0) + jnp.where(iota > q2, 1, 0)
                jw = iota - sh
                g = lax.gather(
                    si, jw[:, None],
                    lax.GatherDimensionNumbers(offset_dims=(),
                                               collapsed_slice_dims=(0,),
                                               start_index_map=(0,)),
                    (1,), mode=lax.GatherScatterMode.PROMISE_IN_BOUNDS)
                res = jnp.where(iota == q1, lb1,
                                jnp.where(iota == q2, lb0, g))
                ov[i, :] = res
                return __

            lax.fori_loop(0, CH, row_body, 0, unroll=4)
            pltpu.sync_copy(ov, out_hbm.at[pl.ds(base, CH)])
            return _

        lax.fori_loop(0, n_chunks, chunk_body, 0)

    return k


def kernel(Q, K, Wq, Wk, logit_scale, block_size, selected_blocks, groups):
    B, T, HQ, D = Q.shape
    H = K.shape[2]
    S = Wq.shape[2]
    NB = T // D

    s = _tc_scores(Q, K, Wq, Wk, logit_scale)
    s2 = s.reshape(B * H * T, NB)
    out2 = _sc_router(s2, T, NB, S)(s2)
    out = out2.reshape(B, H, T, S)
    return jnp.transpose(out, (0, 2, 1, 3))
